# baseline (speedup 1.0000x reference)
import functools

import jax
import jax.numpy as jnp
from jax import lax
from jax.experimental import pallas as pl
from jax.experimental.pallas import tpu as pltpu
from jax.experimental.pallas import tpu_sc as plsc

_GRP = 4
_BLK = 16 * _GRP


def _round_bf16(v):
    u = lax.bitcast_convert_type(v, jnp.uint32)
    lsb = (u >> jnp.uint32(16)) & jnp.uint32(1)
    u = (u + jnp.uint32(32767) + lsb) & jnp.uint32(0xFFFF0000)
    return lax.bitcast_convert_type(u, jnp.float32)


def _make_sc_call(b, s, d, t):
    n = b * s
    info = plsc.get_sparse_core_info()
    nc, ns = info.num_cores, info.num_subcores
    nw = nc * ns
    tpw = n // nw
    spw = s // (nw // b)
    nblk = tpw // _BLK
    mesh = plsc.VectorSubcoreMesh(core_axis_name="c", subcore_axis_name="s")

    @functools.partial(
        pl.kernel, mesh=mesh,
        out_type=[jax.ShapeDtypeStruct((b, s), jnp.int32),
                  jax.ShapeDtypeStruct((b, s), jnp.int32)],
        scratch_types=[
            pltpu.VMEM((d, tpw), jnp.float32),
            pltpu.VMEM((tpw,), jnp.int32),
            pltpu.VMEM((d, 8 * 16), jnp.float32),
            pltpu.VMEM((16,), jnp.int32),
            pltpu.VMEM((tpw,), jnp.int32),
            pltpu.VMEM((tpw,), jnp.int32),
        ],
    )
    def sc_router(xt_hbm, pos_hbm, w_hbm, half_hbm, sel_hbm, tgt_hbm,
                  xv, posv, wv, halfv, selv, tgtv):
        wid = lax.axis_index("s") * nc + lax.axis_index("c")
        bi = wid // (nw // b)
        s0 = (wid % (nw // b)) * spw
        pltpu.sync_copy(xt_hbm.at[bi, :, pl.ds(s0, tpw)], xv)
        pltpu.sync_copy(pos_hbm.at[bi, pl.ds(s0, tpw)], posv)
        pltpu.sync_copy(w_hbm, wv)
        pltpu.sync_copy(half_hbm, halfv)
        halfvec = halfv[...]

        zero = jnp.zeros((16,), jnp.float32)

        def blk_body(blk, carry):
            base = blk * _BLK

            def c_body(c, accs):
                xcs = [
                    _round_bf16(xv[c, pl.ds(base + g * 16, 16)])
                    for g in range(_GRP)
                ]
                out = []
                for ti in range(t):
                    wtc = wv[c, pl.ds(ti * 16, 16)]
                    for g in range(_GRP):
                        out.append(accs[ti * _GRP + g] + xcs[g] * wtc)
                return out

            accs = lax.fori_loop(0, d, c_body, [zero] * (t * _GRP))

            for g in range(_GRP):
                best = accs[g]
                arg = jnp.zeros((16,), jnp.int32)
                for ti in range(1, t):
                    a = accs[ti * _GRP + g]
                    m = a > best
                    best = jnp.where(m, a, best)
                    arg = jnp.where(m, jnp.int32(ti), arg)
                off = base + g * 16
                selv[pl.ds(off, 16)] = arg
                pos = posv[pl.ds(off, 16)]
                x0 = xv[0, pl.ds(off, 16)]
                x1 = xv[1, pl.ds(off, 16)]
                tgtv[pl.ds(off, 16)] = (
                    jnp.where(pos >= halfvec, 4, 0)
                    + jnp.where(x0 > 0, 2, 0)
                    + jnp.where(x1 > 0, 1, 0)).astype(jnp.int32)
            return carry

        lax.fori_loop(0, nblk, blk_body, 0)
        pltpu.sync_copy(selv, sel_hbm.at[bi, pl.ds(s0, tpw)])
        pltpu.sync_copy(tgtv, tgt_hbm.at[bi, pl.ds(s0, tpw)])

    return sc_router


def kernel(x, positions, seq_len, content_sigs):
    b, s, d = x.shape
    t = content_sigs.shape[0]
    w = jnp.tanh(content_sigs)
    wu = lax.bitcast_convert_type(w, jnp.uint32)
    wu = (wu + jnp.uint32(32767) + ((wu >> jnp.uint32(16)) & jnp.uint32(1)))         & jnp.uint32(0xFFFF0000)
    wb = lax.bitcast_convert_type(wu, jnp.float32)
    wsp = jnp.broadcast_to(wb.T[:, :, None], (d, t, 16)).reshape(d, t * 16)
    half = ((jnp.asarray(seq_len) + 1) // 2).astype(jnp.int32)
    halfv = jnp.broadcast_to(half, (16,))
    pos = positions.astype(jnp.int32)
    xt = jnp.transpose(x, (0, 2, 1))
    sel, tgt = _make_sc_call(b, s, d, t)(xt, pos, wsp, halfv)
    return sel, tgt
